# baseline (speedup 1.0000x reference)
import math

import jax
import jax.numpy as jnp
from jax import lax
from jax.experimental import pallas as pl
from jax.experimental.pallas import tpu as pltpu
from jax.experimental.pallas import tpu_sc as plsc

VOCAB = 1000000
D = 64
DP = 128
BATCH = 4096
SEQ = 200
SCALE = math.sqrt(D)

NC = 2
NS = 16
NW = NC * NS
HB = BATCH // 2
SW = HB // NW

TBLK = 8192


def _tpose_in_kernel(t_ref, o_ref):
    o_ref[:, 0:D] = t_ref[...].T


def _tpose_in(tt):
    return pl.pallas_call(
        _tpose_in_kernel,
        grid=(pl.cdiv(VOCAB, TBLK),),
        in_specs=[pl.BlockSpec((D, TBLK), lambda i: (0, i))],
        out_specs=pl.BlockSpec((TBLK, DP), lambda i: (i, 0)),
        out_shape=jax.ShapeDtypeStruct((VOCAB, DP), jnp.float32),
    )(tt)


def _tpose_out_kernel(g_ref, o_ref):
    v = g_ref[0]
    o_ref[0, :, 0:HB] = v[:, 0:D].T * SCALE
    o_ref[0, :, HB:BATCH] = v[:, D:DP].T * SCALE


def _tpose_out(g2):
    return pl.pallas_call(
        _tpose_out_kernel,
        grid=(SEQ,),
        in_specs=[pl.BlockSpec((1, HB, DP), lambda s: (s, 0, 0))],
        out_specs=pl.BlockSpec((1, D, BATCH), lambda s: (s, 0, 0)),
        out_shape=jax.ShapeDtypeStruct((SEQ, D, BATCH), jnp.float32),
    )(g2)


def _embed_kernel(xp_hbm, tpad_hbm, g2_hbm,
                  idx_v, rows_v0, rows_v1, cbuf0, cbuf1,
                  gsem0, gsem1, osem0, osem1):
    wid = lax.axis_index("s") * NC + lax.axis_index("c")
    b0 = wid * 2 * SW
    p0 = wid * SW

    pltpu.sync_copy(xp_hbm.at[:, pl.ds(b0, 2 * SW)], idx_v)

    def fire_gathers(s, rows_vb, gsemb):
        pltpu.async_copy(tpad_hbm.at[idx_v.at[s]], rows_vb, gsemb)

    def wait_gathers(rows_vb, gsemb):
        pltpu.make_async_copy(tpad_hbm.at[pl.ds(0, 2 * SW)], rows_vb,
                              gsemb).wait()

    def compact(rows_vb, cbufb):
        def prow(p, _):
            for j in range(D // 16):
                sl = pl.ds(j * 16, 16)
                cbufb[p, sl] = rows_vb[p, sl]
                cbufb[p, pl.ds(D + j * 16, 16)] = rows_vb[SW + p, sl]
            return 0
        lax.fori_loop(0, SW, prow, 0, unroll=4)

    def fire_store(s, cbufb, osemb):
        pltpu.async_copy(cbufb, g2_hbm.at[s, pl.ds(p0, SW)], osemb)

    def wait_store(cbufb, osemb):
        pltpu.make_async_copy(cbufb, g2_hbm.at[0, pl.ds(p0, SW)], osemb).wait()

    fire_gathers(0, rows_v0, gsem0)

    def pair_body(g, _):
        for b in range(2):
            s = 2 * g + b
            if b == 0:
                cur_rows, cur_c, cur_g, cur_o = rows_v0, cbuf0, gsem0, osem0
                nxt_rows, nxt_c, nxt_g, nxt_o = rows_v1, cbuf1, gsem1, osem1
            else:
                cur_rows, cur_c, cur_g, cur_o = rows_v1, cbuf1, gsem1, osem1
                nxt_rows, nxt_c, nxt_g, nxt_o = rows_v0, cbuf0, gsem0, osem0

            @pl.when(s + 1 < SEQ)
            def _():
                fire_gathers(s + 1, nxt_rows, nxt_g)

            wait_gathers(cur_rows, cur_g)

            @pl.when(s >= 2)
            def _():
                wait_store(cur_c, cur_o)
            compact(cur_rows, cur_c)
            fire_store(s, cur_c, cur_o)
        return 0

    lax.fori_loop(0, SEQ // 2, pair_body, 0)
    wait_store(cbuf0, osem0)
    wait_store(cbuf1, osem1)


@jax.jit
def kernel(x, table):
    tt = jnp.transpose(table)
    xt = jnp.transpose(x).astype(jnp.int32)
    xp = jnp.concatenate(
        [xt[:, :HB].reshape(SEQ, NW, SW), xt[:, HB:].reshape(SEQ, NW, SW)],
        axis=2,
    ).reshape(SEQ, BATCH)
    tpad = _tpose_in(tt)
    mesh = plsc.VectorSubcoreMesh(
        core_axis_name="c", subcore_axis_name="s", num_cores=NC, num_subcores=NS
    )
    g2 = pl.kernel(
        _embed_kernel,
        out_type=jax.ShapeDtypeStruct((SEQ, HB, DP), jnp.float32),
        mesh=mesh,
        scratch_types=[
            pltpu.VMEM((SEQ, 2 * SW), jnp.int32),
            pltpu.VMEM((2 * SW, DP), jnp.float32),
            pltpu.VMEM((2 * SW, DP), jnp.float32),
            pltpu.VMEM((SW, DP), jnp.float32),
            pltpu.VMEM((SW, DP), jnp.float32),
            pltpu.SemaphoreType.DMA,
            pltpu.SemaphoreType.DMA,
            pltpu.SemaphoreType.DMA,
            pltpu.SemaphoreType.DMA,
        ],
        compiler_params=pltpu.CompilerParams(needs_layout_passes=False),
    )(xp, tpad)
    out_t = _tpose_out(g2)
    return jnp.transpose(out_t, (2, 0, 1))


# speedup vs baseline: 1.0739x; 1.0739x over previous
import math

import jax
import jax.numpy as jnp
from jax import lax
from jax.experimental import pallas as pl
from jax.experimental.pallas import tpu as pltpu
from jax.experimental.pallas import tpu_sc as plsc

VOCAB = 1000000
D = 64
DP = 128
BATCH = 4096
SEQ = 200
SCALE = math.sqrt(D)

NC = 2
NS = 16
NW = NC * NS
BW_ = BATCH // NW

TBLK = 8192


def _tpose_in_kernel(t_ref, o_ref):
    o_ref[:, 0:D] = t_ref[...].T


def _tpose_in(tt):
    return pl.pallas_call(
        _tpose_in_kernel,
        grid=(pl.cdiv(VOCAB, TBLK),),
        in_specs=[pl.BlockSpec((D, TBLK), lambda i: (0, i))],
        out_specs=pl.BlockSpec((TBLK, DP), lambda i: (i, 0)),
        out_shape=jax.ShapeDtypeStruct((VOCAB, DP), jnp.float32),
    )(tt)


def _tpose_out_kernel(g_ref, o_ref):
    v = g_ref[0]
    o_ref[0] = v[:, 0:D].T * SCALE


def _tpose_out(g3):
    return pl.pallas_call(
        _tpose_out_kernel,
        grid=(SEQ,),
        in_specs=[pl.BlockSpec((1, BATCH, DP), lambda s: (s, 0, 0))],
        out_specs=pl.BlockSpec((1, D, BATCH), lambda s: (s, 0, 0)),
        out_shape=jax.ShapeDtypeStruct((SEQ, D, BATCH), jnp.float32),
    )(g3)


def _embed_kernel(xt_hbm, tpad_hbm, g3_hbm,
                  idx_v, rows_v0, rows_v1,
                  gsem0, gsem1, osem0, osem1):
    wid = lax.axis_index("s") * NC + lax.axis_index("c")
    b0 = wid * BW_

    pltpu.sync_copy(xt_hbm.at[:, pl.ds(b0, BW_)], idx_v)

    def fire_gather(s, rows_vb, gsemb):
        pltpu.async_copy(tpad_hbm.at[idx_v.at[s]], rows_vb, gsemb)

    def wait_gather(rows_vb, gsemb):
        pltpu.make_async_copy(tpad_hbm.at[pl.ds(0, BW_)], rows_vb, gsemb).wait()

    def fire_store(s, rows_vb, osemb):
        pltpu.async_copy(rows_vb, g3_hbm.at[s, pl.ds(b0, BW_)], osemb)

    def wait_store(rows_vb, osemb):
        pltpu.make_async_copy(rows_vb, g3_hbm.at[0, pl.ds(b0, BW_)],
                              osemb).wait()

    fire_gather(0, rows_v0, gsem0)

    def pair_body(g, _):
        for b in range(2):
            s = 2 * g + b
            if b == 0:
                cur_rows, cur_g, cur_o = rows_v0, gsem0, osem0
                nxt_rows, nxt_g, nxt_o = rows_v1, gsem1, osem1
            else:
                cur_rows, cur_g, cur_o = rows_v1, gsem1, osem1
                nxt_rows, nxt_g, nxt_o = rows_v0, gsem0, osem0

            @pl.when(s + 1 < SEQ)
            def _():
                @pl.when(s >= 1)
                def _():
                    wait_store(nxt_rows, nxt_o)
                fire_gather(s + 1, nxt_rows, nxt_g)

            wait_gather(cur_rows, cur_g)
            fire_store(s, cur_rows, cur_o)
        return 0

    lax.fori_loop(0, SEQ // 2, pair_body, 0)
    wait_store(rows_v0, osem0)
    wait_store(rows_v1, osem1)


@jax.jit
def kernel(x, table):
    tt = jnp.transpose(table)
    xt = jnp.transpose(x).astype(jnp.int32)
    tpad = _tpose_in(tt)
    mesh = plsc.VectorSubcoreMesh(
        core_axis_name="c", subcore_axis_name="s", num_cores=NC, num_subcores=NS
    )
    g3 = pl.kernel(
        _embed_kernel,
        out_type=jax.ShapeDtypeStruct((SEQ, BATCH, DP), jnp.float32),
        mesh=mesh,
        scratch_types=[
            pltpu.VMEM((SEQ, BW_), jnp.int32),
            pltpu.VMEM((BW_, DP), jnp.float32),
            pltpu.VMEM((BW_, DP), jnp.float32),
            pltpu.SemaphoreType.DMA,
            pltpu.SemaphoreType.DMA,
            pltpu.SemaphoreType.DMA,
            pltpu.SemaphoreType.DMA,
        ],
    )(xt, tpad)
    out_t = _tpose_out(g3)
    return jnp.transpose(out_t, (2, 0, 1))
